# baseline (speedup 1.0000x reference)
import functools

import jax
import jax.numpy as jnp
from jax import lax
from jax.experimental import pallas as pl
from jax.experimental.pallas import tpu as pltpu, tpu_sc as plsc

B = 16384
D_IN = 256
NUM_TOKENS = 16
TOKEN_DIM = 64
LANES = 16

NC = 2
NS = 16
NW = NC * NS
B_PER_W = B // NW
CHUNK = 64
N_CHUNKS = B_PER_W // CHUNK


_GATHER_DNUMS = lax.GatherDimensionNumbers(
    offset_dims=(), collapsed_slice_dims=(0,), start_index_map=(0,))


def _bcast_lane(vec, t):
    idx = jnp.full((LANES, 1), t, jnp.int32)
    return lax.gather(vec, idx, _GATHER_DNUMS, (1,),
                      mode=lax.GatherScatterMode.PROMISE_IN_BOUNDS)


def _alpha_body(x_ref, w_ref, b_ref, o_ref):
    att = jnp.dot(x_ref[...], w_ref[...], preferred_element_type=jnp.float32)
    o_ref[...] = jax.nn.sigmoid(att + b_ref[...])


def _compute_alpha(x, W_att, b_att):
    blk = 1024
    return pl.pallas_call(
        _alpha_body,
        grid=(B // blk,),
        in_specs=[
            pl.BlockSpec((blk, D_IN), lambda i: (i, 0)),
            pl.BlockSpec((D_IN, NUM_TOKENS), lambda i: (0, 0)),
            pl.BlockSpec((1, NUM_TOKENS), lambda i: (0, 0)),
        ],
        out_specs=pl.BlockSpec((blk, NUM_TOKENS), lambda i: (i, 0)),
        out_shape=jax.ShapeDtypeStruct((B, NUM_TOKENS), jnp.float32),
    )(x, W_att, b_att.reshape(1, NUM_TOKENS))


def _sc_body(tok_hbm, bias_hbm, idx_hbm, alpha_hbm, out_hbm,
             idx_v, alpha_v, tok_v, bias_v, out_v, sem_t, sem_b):
    wid = lax.axis_index("s") * NC + lax.axis_index("c")
    base = wid * B_PER_W
    pltpu.sync_copy(idx_hbm.at[pl.ds(base, B_PER_W)], idx_v)
    pltpu.sync_copy(alpha_hbm.at[pl.ds(base, B_PER_W)], alpha_v)

    def chunk_body(g, carry):
        goff = g * CHUNK
        tcp = pltpu.async_copy(tok_hbm.at[idx_v.at[pl.ds(goff, CHUNK)]], tok_v, sem_t)
        bcp = pltpu.async_copy(bias_hbm.at[idx_v.at[pl.ds(goff, CHUNK)]], bias_v, sem_b)
        tcp.wait()
        bcp.wait()

        def row_body(r, c2):
            row = goff + r
            a_vec = alpha_v[row, :]
            a = [_bcast_lane(a_vec, t) for t in range(NUM_TOKENS)]
            for c in range(TOKEN_DIM // LANES):
                acc = bias_v[r, pl.ds(c * LANES, LANES)]
                for t in range(NUM_TOKENS):
                    acc = acc + a[t] * tok_v[r, pl.ds(t * TOKEN_DIM + c * LANES, LANES)]
                out_v[r, pl.ds(c * LANES, LANES)] = acc
            return c2

        lax.fori_loop(0, CHUNK, row_body, 0)
        pltpu.sync_copy(out_v, out_hbm.at[pl.ds(base + goff, CHUNK)])
        return carry

    lax.fori_loop(0, N_CHUNKS, chunk_body, 0)


_sc_combine = functools.partial(
    pl.kernel,
    out_type=jax.ShapeDtypeStruct((B, TOKEN_DIM), jnp.float32),
    mesh=plsc.VectorSubcoreMesh(core_axis_name="c", subcore_axis_name="s"),
    scratch_types=[
        pltpu.VMEM((B_PER_W,), jnp.int32),
        pltpu.VMEM((B_PER_W, NUM_TOKENS), jnp.float32),
        pltpu.VMEM((CHUNK, NUM_TOKENS * TOKEN_DIM), jnp.float32),
        pltpu.VMEM((CHUNK, TOKEN_DIM), jnp.float32),
        pltpu.VMEM((CHUNK, TOKEN_DIM), jnp.float32),
        pltpu.SemaphoreType.DMA,
        pltpu.SemaphoreType.DMA,
    ],
    compiler_params=pltpu.CompilerParams(use_tc_tiling_on_sc=False),
)(_sc_body)


def kernel(x, m, style_tokens, style_tokens_bias, W_att, b_att):
    alpha = _compute_alpha(x, W_att, b_att)
    return _sc_combine(style_tokens, style_tokens_bias,
                       m.astype(jnp.int32), alpha)


# speedup vs baseline: 2.7886x; 2.7886x over previous
import functools

import jax
import jax.numpy as jnp
from jax import lax
from jax.experimental import pallas as pl
from jax.experimental.pallas import tpu as pltpu, tpu_sc as plsc

B = 16384
D_IN = 256
NUM_VALUES = 100000
NUM_TOKENS = 16
TOKEN_DIM = 64
LANES = 16

NC = 2
NS = 16
NW = NC * NS
B_PER_W = B // NW
CHUNK = 32
N_CHUNKS = B_PER_W // CHUNK

_GATHER_DNUMS = lax.GatherDimensionNumbers(
    offset_dims=(), collapsed_slice_dims=(0,), start_index_map=(0,))


def _bcast_lane(vec, t):
    idx = jnp.full((LANES, 1), t, jnp.int32)
    return lax.gather(vec, idx, _GATHER_DNUMS, (1,),
                      mode=lax.GatherScatterMode.PROMISE_IN_BOUNDS)


def _alpha_body(x_ref, w_ref, b_ref, o_ref):
    att = jnp.dot(x_ref[...], w_ref[...], preferred_element_type=jnp.float32)
    o_ref[...] = jax.nn.sigmoid(att + b_ref[...])


def _compute_alpha(x, W_att, b_att):
    blk = 1024
    return pl.pallas_call(
        _alpha_body,
        grid=(B // blk,),
        in_specs=[
            pl.BlockSpec((blk, D_IN), lambda i: (i, 0)),
            pl.BlockSpec((D_IN, NUM_TOKENS), lambda i: (0, 0)),
            pl.BlockSpec((1, NUM_TOKENS), lambda i: (0, 0)),
        ],
        out_specs=pl.BlockSpec((blk, NUM_TOKENS), lambda i: (i, 0)),
        out_shape=jax.ShapeDtypeStruct((B, NUM_TOKENS), jnp.float32),
    )(x, W_att, b_att.reshape(1, NUM_TOKENS))


def _sc_body(tok_hbm, bias2_hbm, idx_hbm, alpha_hbm, out_hbm,
             idx_v, idx2_v, par_v,
             tok_v0, tok_v1, bias_v0, bias_v1, al_v0, al_v1, out_v0, out_v1,
             sem_t0, sem_t1, sem_b0, sem_b1, sem_a0, sem_a1, sem_o0, sem_o1):
    wid = lax.axis_index("s") * NC + lax.axis_index("c")
    base = wid * B_PER_W
    pltpu.sync_copy(idx_hbm.at[pl.ds(base, B_PER_W)], idx_v)

    for i in range(B_PER_W // LANES):
        v = idx_v[pl.ds(i * LANES, LANES)]
        idx2_v[pl.ds(i * LANES, LANES)] = lax.shift_right_logical(v, 1)
        par_v[pl.ds(i * LANES, LANES)] = lax.convert_element_type(
            v & 1, jnp.float32)

    tok_bufs = (tok_v0, tok_v1)
    bias_bufs = (bias_v0, bias_v1)
    al_bufs = (al_v0, al_v1)
    out_bufs = (out_v0, out_v1)
    sems_t = (sem_t0, sem_t1)
    sems_b = (sem_b0, sem_b1)
    sems_a = (sem_a0, sem_a1)
    sems_o = (sem_o0, sem_o1)

    def start(g):
        bi = g % 2
        t = pltpu.async_copy(tok_hbm.at[idx_v.at[pl.ds(g * CHUNK, CHUNK)]],
                             tok_bufs[bi], sems_t[bi])
        bcp = pltpu.async_copy(bias2_hbm.at[idx2_v.at[pl.ds(g * CHUNK, CHUNK)]],
                               bias_bufs[bi], sems_b[bi])
        acp = pltpu.async_copy(alpha_hbm.at[pl.ds(base + g * CHUNK, CHUNK)],
                               al_bufs[bi], sems_a[bi])
        return t, bcp, acp

    pending = {0: start(0)}
    out_pending = {}
    for g in range(N_CHUNKS):
        if g + 1 < N_CHUNKS:
            pending[g + 1] = start(g + 1)
        tcp, bcp, acp = pending.pop(g)
        tcp.wait()
        bcp.wait()
        acp.wait()
        if g - 2 in out_pending:
            out_pending.pop(g - 2).wait()
        bi = g % 2
        tok_v = tok_bufs[bi]
        bias_v = bias_bufs[bi]
        al_v = al_bufs[bi]
        out_v = out_bufs[bi]
        goff = g * CHUNK

        def row_body(r, c2, goff=goff, tok_v=tok_v, bias_v=bias_v,
                     al_v=al_v, out_v=out_v):
            row = goff + r
            a_vec = al_v[r, :]
            a = [_bcast_lane(a_vec, t) for t in range(NUM_TOKENS)]
            pv = par_v[pl.ds(row & ~(LANES - 1), LANES)]
            p = _bcast_lane(pv, row & (LANES - 1))
            for c in range(TOKEN_DIM // LANES):
                lo = bias_v[r, pl.ds(c * LANES, LANES)]
                hi = bias_v[r, pl.ds(TOKEN_DIM + c * LANES, LANES)]
                acc = lo + p * (hi - lo)
                for t in range(NUM_TOKENS):
                    acc = acc + a[t] * tok_v[r, pl.ds(t * TOKEN_DIM + c * LANES, LANES)]
                out_v[r, pl.ds(c * LANES, LANES)] = acc
            return c2

        lax.fori_loop(0, CHUNK, row_body, 0)
        out_pending[g] = pltpu.async_copy(
            out_v, out_hbm.at[pl.ds(base + goff, CHUNK)], sems_o[bi])

    for g in sorted(out_pending):
        out_pending.pop(g).wait()


_sc_combine = functools.partial(
    pl.kernel,
    out_type=jax.ShapeDtypeStruct((B, TOKEN_DIM), jnp.float32),
    mesh=plsc.VectorSubcoreMesh(core_axis_name="c", subcore_axis_name="s"),
    scratch_types=[
        pltpu.VMEM((B_PER_W,), jnp.int32),
        pltpu.VMEM((B_PER_W,), jnp.int32),
        pltpu.VMEM((B_PER_W,), jnp.float32),
        pltpu.VMEM((CHUNK, NUM_TOKENS * TOKEN_DIM), jnp.float32),
        pltpu.VMEM((CHUNK, NUM_TOKENS * TOKEN_DIM), jnp.float32),
        pltpu.VMEM((CHUNK, 2 * TOKEN_DIM), jnp.float32),
        pltpu.VMEM((CHUNK, 2 * TOKEN_DIM), jnp.float32),
        pltpu.VMEM((CHUNK, NUM_TOKENS), jnp.float32),
        pltpu.VMEM((CHUNK, NUM_TOKENS), jnp.float32),
        pltpu.VMEM((CHUNK, TOKEN_DIM), jnp.float32),
        pltpu.VMEM((CHUNK, TOKEN_DIM), jnp.float32),
        pltpu.SemaphoreType.DMA,
        pltpu.SemaphoreType.DMA,
        pltpu.SemaphoreType.DMA,
        pltpu.SemaphoreType.DMA,
        pltpu.SemaphoreType.DMA,
        pltpu.SemaphoreType.DMA,
        pltpu.SemaphoreType.DMA,
        pltpu.SemaphoreType.DMA,
    ],
)(_sc_body)


def kernel(x, m, style_tokens, style_tokens_bias, W_att, b_att):
    alpha = _compute_alpha(x, W_att, b_att)
    bias2 = style_tokens_bias.reshape(NUM_VALUES // 2, 2 * TOKEN_DIM)
    return _sc_combine(style_tokens, bias2, m.astype(jnp.int32), alpha)
